# baseline (speedup 1.0000x reference)
import functools

import jax
import jax.numpy as jnp
from jax import lax
from jax.experimental import pallas as pl
from jax.experimental.pallas import tpu as pltpu
from jax.experimental.pallas import tpu_sc as plsc

_BB = 128
_NBUF = 2


@functools.cache
def _build(T: int, B: int, dim: int):
    mesh = plsc.VectorSubcoreMesh(core_axis_name="c", subcore_axis_name="s")
    num_workers = mesh.num_cores * mesh.num_subcores
    nc = mesh.num_cores
    chunks_total = T * (B // _BB)
    chunks_per_worker = chunks_total // num_workers
    b_chunks = B // _BB

    @functools.partial(
        pl.kernel,
        out_type=jax.ShapeDtypeStruct((T, dim, B), jnp.float32),
        mesh=mesh,
        scratch_types=[
            pltpu.VMEM((_BB,), jnp.int32),
            pltpu.VMEM((_BB,), jnp.int32),
            pltpu.VMEM((_BB,), jnp.int32),
            pltpu.VMEM((_BB,), jnp.int32),
            pltpu.VMEM((_BB, 2 * dim), jnp.float32),
            pltpu.VMEM((_BB, 2 * dim), jnp.float32),
            pltpu.VMEM((dim, _BB), jnp.float32),
            pltpu.VMEM((dim, _BB), jnp.float32),
            pltpu.VMEM((_BB,), jnp.int32),
            pltpu.SemaphoreType.DMA,
            pltpu.SemaphoreType.DMA,
            pltpu.SemaphoreType.DMA,
            pltpu.SemaphoreType.DMA,
        ],
        compiler_params=pltpu.CompilerParams(
            use_tc_tiling_on_sc=True, needs_layout_passes=False),
    )
    def body(tok_hbm, tab_hbm, out_hbm,
             tok0, tok1, idx0, idx1, g0, g1, o0, o1, pv, gs0, gs1, ss0, ss1):
        bufs = ((tok0, idx0, g0, o0, gs0, ss0), (tok1, idx1, g1, o1, gs1, ss1))
        wid = lax.axis_index("s") * nc + lax.axis_index("c")
        chunk_base = wid * chunks_per_worker
        iota16 = jax.lax.iota(jnp.int32, 16)

        @pl.loop(0, chunks_per_worker, step=_NBUF)
        def _outer(k0):
            descs = []
            for b in range(_NBUF):
                tok_v, idx_v, g_v, o_v, gsem, ssem = bufs[b]
                g = chunk_base + k0 + b
                t = g // b_chunks
                b0 = (g % b_chunks) * _BB

                @pl.when(k0 > 0)
                def _drain():
                    pltpu.make_async_copy(
                        o_v, out_hbm.at[0, :, pl.ds(0, _BB)], ssem).wait()

                pltpu.sync_copy(tok_hbm.at[t, pl.ds(b0, _BB)], tok_v)
                for j0 in range(0, _BB, 16):
                    idx_v[pl.ds(j0, 16)] = (
                        tok_v[pl.ds(j0, 16)] >> jnp.int32(1))
                descs.append(pltpu.async_copy(tab_hbm.at[idx_v], g_v, gsem))
            for b in range(_NBUF):
                tok_v, idx_v, g_v, o_v, gsem, ssem = bufs[b]
                g = chunk_base + k0 + b
                t = g // b_chunks
                b0 = (g % b_chunks) * _BB
                descs[b].wait()
                for j0 in range(0, _BB, 16):
                    pv[pl.ds(j0, 16)] = (
                        tok_v[pl.ds(j0, 16)] & jnp.int32(1)) * jnp.int32(dim)

                @plsc.parallel_loop(0, (_BB // 16) * dim, unroll=16)
                def _i(i, g_v=g_v, o_v=o_v):
                    grp = i >> 6
                    c = i & (dim - 1)
                    rows = iota16 + (grp << 4)
                    par = pv[pl.ds(grp * 16, 16)]
                    o_v[c, pl.ds(grp * 16, 16)] = plsc.load_gather(
                        g_v, [rows, par + c])
                pltpu.async_copy(o_v, out_hbm.at[t, :, pl.ds(b0, _BB)], ssem)

        for b in range(_NBUF):
            _, _, _, o_v, _, ssem = bufs[b]
            pltpu.make_async_copy(
                o_v, out_hbm.at[0, :, pl.ds(0, _BB)], ssem).wait()

    return body


def kernel(token_ids, weight):
    n_tokens, seq = token_ids.shape
    n_rows, dim = weight.shape
    tok_t = token_ids.T.astype(jnp.int32)
    tab = weight.reshape(n_rows // 2, 2 * dim)
    out = _build(seq, n_tokens, dim)(tok_t, tab)
    return out.transpose(2, 0, 1)


# speedup vs baseline: 1.0410x; 1.0410x over previous
import functools

import jax
import jax.numpy as jnp
from jax import lax
from jax.experimental import pallas as pl
from jax.experimental.pallas import tpu as pltpu
from jax.experimental.pallas import tpu_sc as plsc

_LANES = 128
_G = 4
_CHUNK = _LANES * _G
_NBUF = 2
_BW = 2048


@functools.cache
def _build_gather(num_idx_rows: int, dim: int):
    mesh = plsc.VectorSubcoreMesh(core_axis_name="c", subcore_axis_name="s")
    num_workers = mesh.num_cores * mesh.num_subcores
    rows_per_worker = num_idx_rows // num_workers
    chunks = rows_per_worker // _G
    nc = mesh.num_cores

    @functools.partial(
        pl.kernel,
        out_type=jax.ShapeDtypeStruct((num_idx_rows * _LANES, dim), jnp.float32),
        mesh=mesh,
        scratch_types=[
            pltpu.VMEM((_G, _LANES), jnp.int32),
            pltpu.VMEM((_G, _LANES), jnp.int32),
            pltpu.VMEM((_CHUNK, dim), jnp.float32),
            pltpu.VMEM((_CHUNK, dim), jnp.float32),
            pltpu.SemaphoreType.DMA,
            pltpu.SemaphoreType.DMA,
            pltpu.SemaphoreType.DMA,
            pltpu.SemaphoreType.DMA,
        ],
        compiler_params=pltpu.CompilerParams(use_tc_tiling_on_sc=False),
    )
    def body(idx_hbm, table_hbm, out_hbm, idx0, idx1, rows0, rows1,
             g0, g1, s0, s1):
        bufs = ((idx0, rows0, g0, s0), (idx1, rows1, g1, s1))
        wid = lax.axis_index("s") * nc + lax.axis_index("c")
        row_base = wid * rows_per_worker

        @pl.loop(0, chunks * _G, step=_NBUF * _G)
        def _outer(c0):
            descs = []
            for b in range(_NBUF):
                idx_v, rows_v, gsem, ssem = bufs[b]
                r0 = row_base + c0 + b * _G

                @pl.when(c0 > 0)
                def _drain():
                    pltpu.make_async_copy(
                        rows_v, out_hbm.at[pl.ds(0, _CHUNK)], ssem).wait()

                pltpu.sync_copy(idx_hbm.at[pl.ds(r0, _G)], idx_v)
                descs.append([
                    pltpu.async_copy(
                        table_hbm.at[idx_v.at[j]],
                        rows_v.at[pl.ds(j * _LANES, _LANES)],
                        gsem,
                    )
                    for j in range(_G)
                ])
            for b in range(_NBUF):
                idx_v, rows_v, gsem, ssem = bufs[b]
                for d in descs[b]:
                    d.wait()
                r0 = row_base + c0 + b * _G
                pltpu.async_copy(
                    rows_v, out_hbm.at[pl.ds(r0 * _LANES, _CHUNK)], ssem)

        for b in range(_NBUF):
            _, rows_v, _, ssem = bufs[b]
            pltpu.make_async_copy(
                rows_v, out_hbm.at[pl.ds(0, _CHUNK)], ssem).wait()

    return body


def _tc_transpose_body(x_ref, o_ref):
    o_ref[0] = jnp.transpose(x_ref[...], (1, 0))


@functools.cache
def _build_transpose(T: int, B: int, dim: int):
    nb = B // _BW
    return pl.pallas_call(
        _tc_transpose_body,
        grid=(T, nb),
        in_specs=[pl.BlockSpec((_BW, dim), lambda t, bb: (t * nb + bb, 0))],
        out_specs=pl.BlockSpec((1, dim, _BW), lambda t, bb: (t, 0, bb)),
        out_shape=jax.ShapeDtypeStruct((T, dim, B), jnp.float32),
    )


def kernel(token_ids, weight):
    n_tokens, seq = token_ids.shape
    dim = weight.shape[1]
    idx2d = token_ids.T.astype(jnp.int32).reshape(-1, _LANES)
    flat = _build_gather(idx2d.shape[0], dim)(idx2d, weight)
    out = _build_transpose(seq, n_tokens, dim)(flat)
    return out.transpose(2, 0, 1)


# speedup vs baseline: 1.5645x; 1.5028x over previous
import functools

import jax
import jax.numpy as jnp
from jax import lax
from jax.experimental import pallas as pl
from jax.experimental.pallas import tpu as pltpu
from jax.experimental.pallas import tpu_sc as plsc

_BB = 128
_NBUF = 2


@functools.cache
def _build(T: int, B: int, dim: int):
    mesh = plsc.VectorSubcoreMesh(core_axis_name="c", subcore_axis_name="s")
    num_workers = mesh.num_cores * mesh.num_subcores
    nc = mesh.num_cores
    chunks_total = T * (B // _BB)
    chunks_per_worker = chunks_total // num_workers
    b_chunks = B // _BB

    @functools.partial(
        pl.kernel,
        out_type=jax.ShapeDtypeStruct((T, dim, B), jnp.float32),
        mesh=mesh,
        scratch_types=[
            pltpu.VMEM((_BB,), jnp.int32),
            pltpu.VMEM((_BB,), jnp.int32),
            pltpu.VMEM((_BB,), jnp.int32),
            pltpu.VMEM((_BB,), jnp.int32),
            pltpu.VMEM((_BB, 2 * dim), jnp.float32),
            pltpu.VMEM((_BB, 2 * dim), jnp.float32),
            pltpu.VMEM((dim, _BB), jnp.float32),
            pltpu.VMEM((dim, _BB), jnp.float32),
            pltpu.VMEM((_BB,), jnp.int32),
            pltpu.SemaphoreType.DMA,
            pltpu.SemaphoreType.DMA,
            pltpu.SemaphoreType.DMA,
            pltpu.SemaphoreType.DMA,
        ],
        compiler_params=pltpu.CompilerParams(
            use_tc_tiling_on_sc=True, needs_layout_passes=False),
    )
    def body(tok_hbm, tab_hbm, out_hbm,
             tok0, tok1, idx0, idx1, g0, g1, o0, o1, pv, gs0, gs1, ss0, ss1):
        bufs = ((tok0, idx0, g0, o0, gs0, ss0), (tok1, idx1, g1, o1, gs1, ss1))
        wid = lax.axis_index("s") * nc + lax.axis_index("c")
        chunk_base = wid * chunks_per_worker
        iota16 = jax.lax.iota(jnp.int32, 16)

        @pl.loop(0, chunks_per_worker, step=_NBUF)
        def _outer(k0):
            descs = []
            for b in range(_NBUF):
                tok_v, idx_v, g_v, o_v, gsem, ssem = bufs[b]
                g = chunk_base + k0 + b
                t = g // b_chunks
                b0 = (g % b_chunks) * _BB

                @pl.when(k0 > 0)
                def _drain():
                    pltpu.make_async_copy(
                        o_v, out_hbm.at[0, :, pl.ds(0, _BB)], ssem).wait()

                pltpu.sync_copy(tok_hbm.at[t, pl.ds(b0, _BB)], tok_v)
                for j0 in range(0, _BB, 16):
                    idx_v[pl.ds(j0, 16)] = (
                        tok_v[pl.ds(j0, 16)] >> jnp.int32(1))
                descs.append(pltpu.async_copy(tab_hbm.at[idx_v], g_v, gsem))
            for b in range(_NBUF):
                tok_v, idx_v, g_v, o_v, gsem, ssem = bufs[b]
                g = chunk_base + k0 + b
                t = g // b_chunks
                b0 = (g % b_chunks) * _BB
                descs[b].wait()
                for j0 in range(0, _BB, 16):
                    pv[pl.ds(j0, 16)] = (
                        tok_v[pl.ds(j0, 16)] & jnp.int32(1)) * jnp.int32(dim)
                for j0 in range(0, _BB, 16):
                    rows = iota16 + jnp.int32(j0)
                    par = pv[pl.ds(j0, 16)]

                    @plsc.parallel_loop(0, dim, unroll=8)
                    def _c(c0, rows=rows, par=par, g_v=g_v, o_v=o_v):
                        cvec = (c0 + iota16) & jnp.int32(dim - 1)
                        plsc.store_scatter(
                            o_v, [cvec, rows],
                            plsc.load_gather(g_v, [rows, par + cvec]))
                pltpu.async_copy(o_v, out_hbm.at[t, :, pl.ds(b0, _BB)], ssem)

        for b in range(_NBUF):
            _, _, _, o_v, _, ssem = bufs[b]
            pltpu.make_async_copy(
                o_v, out_hbm.at[0, :, pl.ds(0, _BB)], ssem).wait()

    return body


def kernel(token_ids, weight):
    n_tokens, seq = token_ids.shape
    n_rows, dim = weight.shape
    tok_t = token_ids.T.astype(jnp.int32)
    tab = weight.reshape(n_rows // 2, 2 * dim)
    out = _build(seq, n_tokens, dim)(tok_t, tab)
    return out.transpose(2, 0, 1)


# speedup vs baseline: 2.2573x; 1.4428x over previous
import functools

import jax
import jax.numpy as jnp
from jax import lax
from jax.experimental import pallas as pl
from jax.experimental.pallas import tpu as pltpu
from jax.experimental.pallas import tpu_sc as plsc

_BB = 128
_NBUF = 2


@functools.cache
def _build(T: int, B: int, dim: int):
    mesh = plsc.VectorSubcoreMesh(core_axis_name="c", subcore_axis_name="s")
    num_workers = mesh.num_cores * mesh.num_subcores
    nc = mesh.num_cores
    chunks_total = T * (B // _BB)
    chunks_per_worker = chunks_total // num_workers
    b_chunks = B // _BB

    @functools.partial(
        pl.kernel,
        out_type=jax.ShapeDtypeStruct((T, dim, B), jnp.float32),
        mesh=mesh,
        scratch_types=[
            pltpu.VMEM((_BB,), jnp.int32),
            pltpu.VMEM((_BB,), jnp.int32),
            pltpu.VMEM((_BB,), jnp.int32),
            pltpu.VMEM((_BB,), jnp.int32),
            pltpu.VMEM((_BB, 2 * dim), jnp.float32),
            pltpu.VMEM((_BB, 2 * dim), jnp.float32),
            pltpu.VMEM((dim, _BB), jnp.float32),
            pltpu.VMEM((dim, _BB), jnp.float32),
            pltpu.VMEM((_BB,), jnp.int32),
            pltpu.SemaphoreType.DMA,
            pltpu.SemaphoreType.DMA,
            pltpu.SemaphoreType.DMA,
            pltpu.SemaphoreType.DMA,
        ],
        compiler_params=pltpu.CompilerParams(
            use_tc_tiling_on_sc=True, needs_layout_passes=False),
    )
    def body(tok_hbm, tab_hbm, out_hbm,
             tok0, tok1, idx0, idx1, g0, g1, o0, o1, pv, gs0, gs1, ss0, ss1):
        bufs = ((tok0, idx0, g0, o0, gs0, ss0), (tok1, idx1, g1, o1, gs1, ss1))
        wid = lax.axis_index("s") * nc + lax.axis_index("c")
        chunk_base = wid * chunks_per_worker
        iota16 = jax.lax.iota(jnp.int32, 16)

        @pl.loop(0, chunks_per_worker, step=_NBUF)
        def _outer(k0):
            descs = []
            for b in range(_NBUF):
                tok_v, idx_v, g_v, o_v, gsem, ssem = bufs[b]
                g = chunk_base + k0 + b
                t = g // b_chunks
                b0 = (g % b_chunks) * _BB

                @pl.when(k0 > 0)
                def _drain():
                    pltpu.make_async_copy(
                        o_v, out_hbm.at[0, :, pl.ds(0, _BB)], ssem).wait()

                pltpu.sync_copy(tok_hbm.at[t, pl.ds(b0, _BB)], tok_v)
                for j0 in range(0, _BB, 16):
                    idx_v[pl.ds(j0, 16)] = (
                        tok_v[pl.ds(j0, 16)] >> jnp.int32(1))
                descs.append(pltpu.async_copy(tab_hbm.at[idx_v], g_v, gsem))
            for b in range(_NBUF):
                tok_v, idx_v, g_v, o_v, gsem, ssem = bufs[b]
                g = chunk_base + k0 + b
                t = g // b_chunks
                b0 = (g % b_chunks) * _BB
                descs[b].wait()
                for j0 in range(0, _BB, 16):
                    pv[pl.ds(j0, 16)] = (
                        tok_v[pl.ds(j0, 16)] & jnp.int32(1)) * jnp.int32(dim)
                for j0 in range(0, _BB, 16):
                    rows = iota16 + jnp.int32(j0)
                    par = pv[pl.ds(j0, 16)]

                    @plsc.parallel_loop(0, dim, unroll=8)
                    def _c(c0, rows=rows, par=par, g_v=g_v, o_v=o_v):
                        cvec = (c0 + iota16) & jnp.int32(dim - 1)
                        plsc.store_scatter(
                            o_v, [cvec, rows],
                            plsc.load_gather(g_v, [rows, par + cvec]))
                pltpu.async_copy(o_v, out_hbm.at[t, :, pl.ds(b0, _BB)], ssem)

        for b in range(_NBUF):
            _, _, _, o_v, _, ssem = bufs[b]
            pltpu.make_async_copy(
                o_v, out_hbm.at[0, :, pl.ds(0, _BB)], ssem).wait()

    return body


@functools.cache
def _build_pairs(n_rows: int, dim: int):
    mesh = plsc.VectorSubcoreMesh(core_axis_name="c", subcore_axis_name="s")
    num_workers = mesh.num_cores * mesh.num_subcores
    nc = mesh.num_cores
    full_blocks = n_rows // _BB
    per_worker = full_blocks // num_workers
    extra = full_blocks - per_worker * num_workers
    tail_rows = (n_rows - full_blocks * _BB) // 2

    @functools.partial(
        pl.kernel,
        out_type=jax.ShapeDtypeStruct((n_rows // 2, 2 * dim), jnp.float32),
        mesh=mesh,
        scratch_types=[
            pltpu.VMEM((dim, _BB), jnp.float32),
            pltpu.VMEM((dim, _BB), jnp.float32),
            pltpu.VMEM((dim, _BB), jnp.float32),
            pltpu.VMEM((dim, _BB), jnp.float32),
            pltpu.SemaphoreType.DMA,
            pltpu.SemaphoreType.DMA,
            pltpu.SemaphoreType.DMA,
            pltpu.SemaphoreType.DMA,
        ],
        compiler_params=pltpu.CompilerParams(
            use_tc_tiling_on_sc=True, needs_layout_passes=False),
    )
    def body(wt_hbm, tail_hbm, w2_hbm, i0, i1, o0, o1, gs0, gs1, ss0, ss1):
        bufs = ((i0, o0, gs0, ss0), (i1, o1, gs1, ss1))
        wid = lax.axis_index("s") * nc + lax.axis_index("c")
        blk_base = wid * per_worker
        iota16 = jax.lax.iota(jnp.int32, 16)

        def transpose_block(v_in, v_out):
            for x0 in range(0, _BB, 16):
                xv = iota16 + jnp.int32(x0)
                rows_out = xv >> jnp.int32(1)
                colbase = (xv & jnp.int32(1)) << jnp.int32(6)

                @plsc.parallel_loop(0, dim, unroll=8)
                def _c(c0, xv=xv, rows_out=rows_out, colbase=colbase,
                       v_in=v_in, v_out=v_out):
                    cv = (c0 + iota16) & jnp.int32(dim - 1)
                    plsc.store_scatter(
                        v_out, [rows_out, colbase + cv],
                        plsc.load_gather(v_in, [cv, xv]))

        @pl.loop(0, per_worker, step=_NBUF)
        def _outer(k0):
            descs = []
            for b in range(_NBUF):
                v_in, v_out, gsem, ssem = bufs[b]
                blk = blk_base + k0 + b

                @pl.when(k0 > 0)
                def _drain():
                    pltpu.make_async_copy(
                        v_out, w2_hbm.at[pl.ds(0, dim)], ssem).wait()

                descs.append(pltpu.async_copy(
                    wt_hbm.at[:, pl.ds(blk * _BB, _BB)], v_in, gsem))
            for b in range(_NBUF):
                v_in, v_out, gsem, ssem = bufs[b]
                blk = blk_base + k0 + b
                descs[b].wait()
                transpose_block(v_in, v_out)
                pltpu.async_copy(
                    v_out, w2_hbm.at[pl.ds(blk * (_BB // 2), dim)], ssem)

        for b in range(_NBUF):
            v_in, v_out, gsem, ssem = bufs[b]
            pltpu.make_async_copy(
                v_out, w2_hbm.at[pl.ds(0, dim)], ssem).wait()

        @pl.when(wid < extra)
        def _extra():
            v_in, v_out, gsem, ssem = bufs[0]
            blk = per_worker * num_workers + wid
            pltpu.async_copy(
                wt_hbm.at[:, pl.ds(blk * _BB, _BB)], v_in, gsem).wait()
            transpose_block(v_in, v_out)
            pltpu.async_copy(
                v_out, w2_hbm.at[pl.ds(blk * (_BB // 2), dim)], ssem).wait()

        @pl.when(wid == num_workers - 1)
        def _tail():
            v_in, _, gsem, _ = bufs[1]
            pltpu.async_copy(
                tail_hbm, v_in.at[pl.ds(0, tail_rows)], gsem).wait()
            pltpu.sync_copy(
                v_in.at[pl.ds(0, tail_rows)],
                w2_hbm.at[pl.ds(full_blocks * (_BB // 2), tail_rows)])

    return body


def kernel(token_ids, weight):
    n_tokens, seq = token_ids.shape
    n_rows, dim = weight.shape
    tok_t = token_ids.T.astype(jnp.int32)
    w_t = weight.T
    n_tail = (n_rows // _BB) * _BB
    tail = weight[n_tail:].reshape(-1, 2 * dim)
    tab = _build_pairs(n_rows, dim)(w_t, tail)
    out = _build(seq, n_tokens, dim)(tok_t, tab)
    return out.transpose(2, 0, 1)
